# Initial kernel scaffold; baseline (speedup 1.0000x reference)
import functools

import jax
import jax.numpy as jnp
from jax import lax
from jax.experimental import pallas as pl
from jax.experimental.pallas import tpu as pltpu
from jax.experimental.pallas import tpu_sc as plsc

N = 10000
E = 320000
D = 128
DC = 64
NC_REAL = 40
NP = 10240
PN = N
NCORE = 2
NSUB = 16
NW = NCORE * NSUB
CH = 128
CCH = 80
EPAD = NW * CCH * CH
RPS = NP // NSUB


def _sc_mesh():
    return plsc.VectorSubcoreMesh(core_axis_name="c", subcore_axis_name="s")


def _make_sc_agg(d):
    @functools.partial(
        pl.kernel,
        out_type=jax.ShapeDtypeStruct((NCORE, NP, d), jnp.float32),
        mesh=_sc_mesh(),
        scratch_types=[
            pltpu.VMEM((CCH, CH), jnp.int32),
            pltpu.VMEM((CCH, CH), jnp.int32),
            pltpu.VMEM((CH, d), jnp.float32),
            pltpu.VMEM_SHARED((NP, d), jnp.float32),
            pltpu.SemaphoreType.DMA,
        ],
    )
    def agg(gs_hbm, src_hbm, dst_hbm, zr_hbm, out_hbm, src_v, dst_v, rows_v,
            acc_sh, sem):
        cid = lax.axis_index("c")
        sid = lax.axis_index("s")
        wid = cid * NSUB + sid
        pltpu.sync_copy(zr_hbm, acc_sh.at[pl.ds(sid * RPS, RPS)])
        pltpu.sync_copy(src_hbm.at[wid], src_v)
        pltpu.sync_copy(dst_hbm.at[wid], dst_v)
        plsc.subcore_barrier()

        def body(j, carry):
            pltpu.async_copy(gs_hbm.at[src_v.at[j]], rows_v, sem).wait()
            pltpu.sync_copy(rows_v, acc_sh.at[dst_v.at[j]], add=True)
            return carry

        lax.fori_loop(0, CCH, body, 0)
        plsc.subcore_barrier()
        pltpu.sync_copy(acc_sh.at[pl.ds(sid * RPS, RPS)],
                        out_hbm.at[cid, pl.ds(sid * RPS, RPS)])

    return agg


_sc_agg_d = _make_sc_agg(D)
_sc_agg_c = _make_sc_agg(DC)


@functools.partial(
    pl.kernel,
    out_type=jax.ShapeDtypeStruct((NCORE, NP, 1), jnp.float32),
    mesh=_sc_mesh(),
    scratch_types=[
        pltpu.VMEM((CCH, CH), jnp.int32),
        pltpu.VMEM((CH, 1), jnp.float32),
        pltpu.VMEM_SHARED((NP, 1), jnp.float32),
    ],
)
def _sc_deg(dst_hbm, ones_hbm, zr_hbm, out_hbm, dst_v, ones_v, acc_sh):
    cid = lax.axis_index("c")
    sid = lax.axis_index("s")
    wid = cid * NSUB + sid
    pltpu.sync_copy(zr_hbm, acc_sh.at[pl.ds(sid * RPS, RPS)])
    pltpu.sync_copy(dst_hbm.at[wid], dst_v)
    pltpu.sync_copy(ones_hbm, ones_v)
    plsc.subcore_barrier()

    def body(j, carry):
        pltpu.sync_copy(ones_v, acc_sh.at[dst_v.at[j]], add=True)
        return carry

    lax.fori_loop(0, CCH, body, 0)
    plsc.subcore_barrier()
    pltpu.sync_copy(acc_sh.at[pl.ds(sid * RPS, RPS)],
                    out_hbm.at[cid, pl.ds(sid * RPS, RPS)])


def _tc0_body(degp_ref, x_ref, w_ref, dinv_ref, gs_ref):
    deg = degp_ref[0] + degp_ref[1]
    rid = lax.broadcasted_iota(jnp.int32, (NP, 1), 0)
    deg = jnp.where(rid < N, deg + 1.0, deg)
    dinv = jnp.where(deg > 0, lax.rsqrt(deg), 0.0)
    dinv_ref[...] = dinv
    h = jnp.dot(x_ref[...], w_ref[...], preferred_element_type=jnp.float32)
    gs_ref[...] = jnp.zeros((NP, D), jnp.float32)
    gs_ref[0:N, :] = h * dinv[0:N]


def _tc0(degp, x, w1):
    return pl.pallas_call(
        _tc0_body,
        out_shape=(jax.ShapeDtypeStruct((NP, 1), jnp.float32),
                   jax.ShapeDtypeStruct((NP, D), jnp.float32)),
    )(degp, x, w1)


def _make_tc_mid(dout, relu):
    def body(aggp_ref, gs_ref, dinv_ref, b_ref, w_ref, out_ref):
        dinv = dinv_ref[...]
        t = aggp_ref[0] + aggp_ref[1] + gs_ref[...]
        h = t * dinv + b_ref[...]
        if relu:
            h = jnp.maximum(h, 0.0)
        g = jnp.dot(h, w_ref[...], preferred_element_type=jnp.float32)
        rid = lax.broadcasted_iota(jnp.int32, (NP, 1), 0)
        out_ref[...] = jnp.where(rid < N, g * dinv, 0.0)

    def call(aggp, gs, dinv, b, w):
        return pl.pallas_call(
            body,
            out_shape=jax.ShapeDtypeStruct((NP, dout), jnp.float32),
        )(aggp, gs, dinv, b, w)

    return call


_tc_mid = _make_tc_mid(D, relu=False)
_tc4 = _make_tc_mid(DC, relu=True)


def _tc5_body(aggp_ref, gs_ref, dinv_ref, b_ref, out_ref):
    t = aggp_ref[0] + aggp_ref[1] + gs_ref[...]
    h = t * dinv_ref[...] + b_ref[...]
    v = h[0:N, 0:NC_REAL]
    m = jnp.max(v, axis=0, keepdims=True)
    e = jnp.exp(v - m)
    out_ref[...] = e / jnp.sum(e, axis=0, keepdims=True)


def _tc5(aggp, gs, dinv, b):
    return pl.pallas_call(
        _tc5_body,
        out_shape=jax.ShapeDtypeStruct((N, NC_REAL), jnp.float32),
    )(aggp, gs, dinv, b)


@jax.jit
def kernel(x, edge_index, W1, b1, Wi, bi, W2, b2):
    src = edge_index[0].astype(jnp.int32)
    dst = edge_index[1].astype(jnp.int32)
    pad = jnp.full((EPAD - E,), PN, jnp.int32)
    src3 = jnp.concatenate([src, pad]).reshape(NW, CCH, CH)
    dst3 = jnp.concatenate([dst, pad]).reshape(NW, CCH, CH)

    zr_d = jnp.zeros((RPS, D), jnp.float32)
    zr_c = jnp.zeros((RPS, DC), jnp.float32)
    zr_1 = jnp.zeros((RPS, 1), jnp.float32)
    ones1 = jnp.ones((CH, 1), jnp.float32)

    w2p = jnp.zeros((D, DC), jnp.float32).at[:, :NC_REAL].set(W2)
    b1r = jnp.reshape(b1, (1, D))
    bir = jnp.reshape(bi, (1, D))
    b2p = jnp.zeros((1, DC), jnp.float32).at[0, :NC_REAL].set(b2)

    degp = _sc_deg(dst3, ones1, zr_1)
    dinv, gs = _tc0(degp, x, W1)

    aggp = _sc_agg_d(gs, src3, dst3, zr_d)
    gs = _tc_mid(aggp, gs, dinv, b1r, Wi)
    for _ in range(2):
        aggp = _sc_agg_d(gs, src3, dst3, zr_d)
        gs = _tc_mid(aggp, gs, dinv, bir, Wi)
    aggp = _sc_agg_d(gs, src3, dst3, zr_d)
    gs = _tc4(aggp, gs, dinv, bir, w2p)

    aggp = _sc_agg_c(gs, src3, dst3, zr_c)
    return _tc5(aggp, gs, dinv, b2p)


# speedup vs baseline: 4.4961x; 4.4961x over previous
import functools

import jax
import jax.numpy as jnp
from jax import lax
from jax.experimental import pallas as pl
from jax.experimental.pallas import tpu as pltpu
from jax.experimental.pallas import tpu_sc as plsc

N = 10000
E = 320000
D = 128
DC = 128
NC_REAL = 40
NP = 10240
PN = N
NCORE = 2
NSUB = 16
NW = NCORE * NSUB
CH = 128
CCH = 80
EPAD = NW * CCH * CH
RPS = NP // NSUB


def _sc_mesh():
    return plsc.VectorSubcoreMesh(core_axis_name="c", subcore_axis_name="s")


def _make_sc_agg(d):
    @functools.partial(
        pl.kernel,
        out_type=jax.ShapeDtypeStruct((NCORE, NP, d), jnp.float32),
        mesh=_sc_mesh(),
        scratch_types=[
            pltpu.VMEM((CCH, CH), jnp.int32),
            pltpu.VMEM((CCH, CH), jnp.int32),
            pltpu.VMEM((CH, d), jnp.float32),
            pltpu.VMEM_SHARED((NP, d), jnp.float32),
            pltpu.SemaphoreType.DMA,
        ],
    )
    def agg(gs_hbm, src_hbm, dst_hbm, zr_hbm, out_hbm, src_v, dst_v, rows_v,
            acc_sh, sem):
        cid = lax.axis_index("c")
        sid = lax.axis_index("s")
        wid = cid * NSUB + sid
        pltpu.sync_copy(zr_hbm, acc_sh.at[pl.ds(sid * RPS, RPS)])
        pltpu.sync_copy(src_hbm.at[wid], src_v)
        pltpu.sync_copy(dst_hbm.at[wid], dst_v)
        plsc.subcore_barrier()

        def body(j, carry):
            pltpu.async_copy(gs_hbm.at[src_v.at[j]], rows_v, sem).wait()
            pltpu.sync_copy(rows_v, acc_sh.at[dst_v.at[j]], add=True)
            return carry

        lax.fori_loop(0, CCH, body, 0)
        plsc.subcore_barrier()
        pltpu.sync_copy(acc_sh.at[pl.ds(sid * RPS, RPS)],
                        out_hbm.at[cid, pl.ds(sid * RPS, RPS)])

    return agg


_sc_agg_d = _make_sc_agg(D)
_sc_agg_c = _make_sc_agg(DC)


def _tc0_body(degp_ref, x_ref, w_ref, dinv_ref, gs_ref):
    deg = degp_ref[0, :, 0:1] + degp_ref[1, :, 0:1]
    rid = lax.broadcasted_iota(jnp.int32, (NP, 1), 0)
    deg = jnp.where(rid < N, deg + 1.0, deg)
    dinv = jnp.where(deg > 0, lax.rsqrt(deg), 0.0)
    dinv_ref[...] = dinv
    h = jnp.dot(x_ref[...], w_ref[...], preferred_element_type=jnp.float32)
    gs_ref[...] = jnp.zeros((NP, D), jnp.float32)
    gs_ref[0:N, :] = h * dinv[0:N]


def _tc0(degp, x, w1):
    return pl.pallas_call(
        _tc0_body,
        out_shape=(jax.ShapeDtypeStruct((NP, 1), jnp.float32),
                   jax.ShapeDtypeStruct((NP, D), jnp.float32)),
    )(degp, x, w1)


def _make_tc_mid(dout, relu):
    def body(aggp_ref, gs_ref, dinv_ref, b_ref, w_ref, out_ref):
        dinv = dinv_ref[...]
        t = aggp_ref[0] + aggp_ref[1] + gs_ref[...]
        h = t * dinv + b_ref[...]
        if relu:
            h = jnp.maximum(h, 0.0)
        g = jnp.dot(h, w_ref[...], preferred_element_type=jnp.float32)
        rid = lax.broadcasted_iota(jnp.int32, (NP, 1), 0)
        out_ref[...] = jnp.where(rid < N, g * dinv, 0.0)

    def call(aggp, gs, dinv, b, w):
        return pl.pallas_call(
            body,
            out_shape=jax.ShapeDtypeStruct((NP, dout), jnp.float32),
        )(aggp, gs, dinv, b, w)

    return call


_tc_mid = _make_tc_mid(D, relu=False)
_tc4 = _make_tc_mid(DC, relu=True)


def _tc5_body(aggp_ref, gs_ref, dinv_ref, b_ref, out_ref):
    t = aggp_ref[0] + aggp_ref[1] + gs_ref[...]
    h = t * dinv_ref[...] + b_ref[...]
    v = h[0:N, 0:NC_REAL]
    m = jnp.max(v, axis=0, keepdims=True)
    e = jnp.exp(v - m)
    out_ref[...] = e / jnp.sum(e, axis=0, keepdims=True)


def _tc5(aggp, gs, dinv, b):
    return pl.pallas_call(
        _tc5_body,
        out_shape=jax.ShapeDtypeStruct((N, NC_REAL), jnp.float32),
    )(aggp, gs, dinv, b)


@jax.jit
def kernel(x, edge_index, W1, b1, Wi, bi, W2, b2):
    src = edge_index[0].astype(jnp.int32)
    dst = edge_index[1].astype(jnp.int32)
    pad = jnp.full((EPAD - E,), PN, jnp.int32)
    src3 = jnp.concatenate([src, pad]).reshape(NW, CCH, CH)
    dst3 = jnp.concatenate([dst, pad]).reshape(NW, CCH, CH)

    zr_d = jnp.zeros((RPS, D), jnp.float32)
    zr_c = jnp.zeros((RPS, DC), jnp.float32)
    ones_m = jnp.ones((NP, D), jnp.float32)

    w2p = jnp.zeros((D, DC), jnp.float32).at[:, :NC_REAL].set(W2)
    b1r = jnp.reshape(b1, (1, D))
    bir = jnp.reshape(bi, (1, D))
    b2p = jnp.zeros((1, DC), jnp.float32).at[0, :NC_REAL].set(b2)

    degp = _sc_agg_d(ones_m, src3, dst3, zr_d)
    dinv, gs = _tc0(degp, x, W1)

    aggp = _sc_agg_d(gs, src3, dst3, zr_d)
    gs = _tc_mid(aggp, gs, dinv, b1r, Wi)
    for _ in range(2):
        aggp = _sc_agg_d(gs, src3, dst3, zr_d)
        gs = _tc_mid(aggp, gs, dinv, bir, Wi)
    aggp = _sc_agg_d(gs, src3, dst3, zr_d)
    gs = _tc4(aggp, gs, dinv, bir, w2p)

    aggp = _sc_agg_c(gs, src3, dst3, zr_c)
    return _tc5(aggp, gs, dinv, b2p)


# speedup vs baseline: 4.8637x; 1.0818x over previous
import functools

import jax
import jax.numpy as jnp
from jax import lax
from jax.experimental import pallas as pl
from jax.experimental.pallas import tpu as pltpu
from jax.experimental.pallas import tpu_sc as plsc

N = 10000
E = 320000
D = 128
DC = 128
NC_REAL = 40
NP = 10240
PN = N
NCORE = 2
NSUB = 16
NW = NCORE * NSUB
CH = 128
CCH = 80
EPAD = NW * CCH * CH
RPS = NP // NSUB


def _sc_mesh():
    return plsc.VectorSubcoreMesh(core_axis_name="c", subcore_axis_name="s")


def _make_sc_agg(d):
    @functools.partial(
        pl.kernel,
        out_type=jax.ShapeDtypeStruct((NCORE, NP, d), jnp.float32),
        mesh=_sc_mesh(),
        scratch_types=[
            pltpu.VMEM((CCH // 2, CH), jnp.int32),
            pltpu.VMEM((CCH // 2, CH), jnp.int32),
            pltpu.VMEM((2, CH, d), jnp.float32),
            pltpu.VMEM_SHARED((NP, d), jnp.float32),
            pltpu.SemaphoreType.DMA,
            pltpu.SemaphoreType.DMA,
        ],
    )
    def agg(gs_hbm, src_hbm, dst_hbm, zr_hbm, out_hbm, src_v, dst_v, rows_v,
            acc_sh, gsem, ssem):
        cid = lax.axis_index("c")
        sid = lax.axis_index("s")
        wid = cid * NSUB + sid
        hl = CCH // 2
        pltpu.sync_copy(zr_hbm, acc_sh.at[pl.ds(sid * RPS, RPS)])
        plsc.subcore_barrier()

        def fire_gather(j, k):
            pltpu.async_copy(gs_hbm.at[src_v.at[j]], rows_v.at[k], gsem)

        def drain_gather():
            pltpu.make_async_copy(gs_hbm.at[pl.ds(0, CH)], rows_v.at[0],
                                  gsem).wait()

        def scatter(j, k):
            pltpu.async_copy(rows_v.at[k], acc_sh.at[dst_v.at[j]],
                             ssem, add=True).wait()

        for h in range(2):
            pltpu.sync_copy(src_hbm.at[wid, pl.ds(h * hl, hl)], src_v)
            pltpu.sync_copy(dst_hbm.at[wid, pl.ds(h * hl, hl)], dst_v)
            fire_gather(0, 0)

            def body(i, carry):
                jb = i * 2
                drain_gather()
                fire_gather(jb + 1, 1)
                scatter(jb, 0)
                drain_gather()
                @pl.when(i < hl // 2 - 1)
                def _():
                    fire_gather(jb + 2, 0)
                scatter(jb + 1, 1)
                return carry

            lax.fori_loop(0, hl // 2, body, 0)

        plsc.subcore_barrier()
        pltpu.sync_copy(acc_sh.at[pl.ds(sid * RPS, RPS)],
                        out_hbm.at[cid, pl.ds(sid * RPS, RPS)])

    return agg


_sc_agg_d = _make_sc_agg(D)
_sc_agg_c = _make_sc_agg(DC)


def _tc0_body(degp_ref, x_ref, w_ref, dinv_ref, gs_ref):
    deg = degp_ref[0, :, 0:1] + degp_ref[1, :, 0:1]
    rid = lax.broadcasted_iota(jnp.int32, (NP, 1), 0)
    deg = jnp.where(rid < N, deg + 1.0, deg)
    dinv = jnp.where(deg > 0, lax.rsqrt(deg), 0.0)
    dinv_ref[...] = dinv
    h = jnp.dot(x_ref[...], w_ref[...], preferred_element_type=jnp.float32)
    gs_ref[...] = jnp.zeros((NP, D), jnp.float32)
    gs_ref[0:N, :] = h * dinv[0:N]


def _tc0(degp, x, w1):
    return pl.pallas_call(
        _tc0_body,
        out_shape=(jax.ShapeDtypeStruct((NP, 1), jnp.float32),
                   jax.ShapeDtypeStruct((NP, D), jnp.float32)),
    )(degp, x, w1)


def _make_tc_mid(dout, relu):
    def body(aggp_ref, gs_ref, dinv_ref, b_ref, w_ref, out_ref):
        dinv = dinv_ref[...]
        t = aggp_ref[0] + aggp_ref[1] + gs_ref[...]
        h = t * dinv + b_ref[...]
        if relu:
            h = jnp.maximum(h, 0.0)
        g = jnp.dot(h, w_ref[...], preferred_element_type=jnp.float32)
        rid = lax.broadcasted_iota(jnp.int32, (NP, 1), 0)
        out_ref[...] = jnp.where(rid < N, g * dinv, 0.0)

    def call(aggp, gs, dinv, b, w):
        return pl.pallas_call(
            body,
            out_shape=jax.ShapeDtypeStruct((NP, dout), jnp.float32),
        )(aggp, gs, dinv, b, w)

    return call


_tc_mid = _make_tc_mid(D, relu=False)
_tc4 = _make_tc_mid(DC, relu=True)


def _tc5_body(aggp_ref, gs_ref, dinv_ref, b_ref, out_ref):
    t = aggp_ref[0] + aggp_ref[1] + gs_ref[...]
    h = t * dinv_ref[...] + b_ref[...]
    v = h[0:N, 0:NC_REAL]
    m = jnp.max(v, axis=0, keepdims=True)
    e = jnp.exp(v - m)
    out_ref[...] = e / jnp.sum(e, axis=0, keepdims=True)


def _tc5(aggp, gs, dinv, b):
    return pl.pallas_call(
        _tc5_body,
        out_shape=jax.ShapeDtypeStruct((N, NC_REAL), jnp.float32),
    )(aggp, gs, dinv, b)


@jax.jit
def kernel(x, edge_index, W1, b1, Wi, bi, W2, b2):
    src = edge_index[0].astype(jnp.int32)
    dst = edge_index[1].astype(jnp.int32)
    pad = jnp.full((EPAD - E,), PN, jnp.int32)
    src3 = jnp.concatenate([src, pad]).reshape(NW, CCH, CH)
    dst3 = jnp.concatenate([dst, pad]).reshape(NW, CCH, CH)

    zr_d = jnp.zeros((RPS, D), jnp.float32)
    zr_c = jnp.zeros((RPS, DC), jnp.float32)
    ones_m = jnp.ones((NP, D), jnp.float32)

    w2p = jnp.zeros((D, DC), jnp.float32).at[:, :NC_REAL].set(W2)
    b1r = jnp.reshape(b1, (1, D))
    bir = jnp.reshape(bi, (1, D))
    b2p = jnp.zeros((1, DC), jnp.float32).at[0, :NC_REAL].set(b2)

    degp = _sc_agg_d(ones_m, src3, dst3, zr_d)
    dinv, gs = _tc0(degp, x, W1)

    aggp = _sc_agg_d(gs, src3, dst3, zr_d)
    gs = _tc_mid(aggp, gs, dinv, b1r, Wi)
    for _ in range(2):
        aggp = _sc_agg_d(gs, src3, dst3, zr_d)
        gs = _tc_mid(aggp, gs, dinv, bir, Wi)
    aggp = _sc_agg_d(gs, src3, dst3, zr_d)
    gs = _tc4(aggp, gs, dinv, bir, w2p)

    aggp = _sc_agg_c(gs, src3, dst3, zr_c)
    return _tc5(aggp, gs, dinv, b2p)


# speedup vs baseline: 6.3720x; 1.3101x over previous
import functools

import jax
import jax.numpy as jnp
from jax import lax
from jax.experimental import pallas as pl
from jax.experimental.pallas import tpu as pltpu
from jax.experimental.pallas import tpu_sc as plsc

N = 10000
E = 320000
D = 128
DC = 128
NC_REAL = 40
NP = 10240
PN = N
NCORE = 2
NSUB = 16
NW = NCORE * NSUB
CH = 128
HL = 32
NST0 = 4
NST1 = 1
CCH0 = NST0 * HL
CCH1 = NST1 * HL
CCHM = max(CCH0, CCH1)
EPAD = NSUB * (CCH0 + CCH1) * CH
E0 = NSUB * CCH0 * CH
RPS = NP // NSUB


def _sc_mesh():
    return plsc.VectorSubcoreMesh(core_axis_name="c", subcore_axis_name="s")


def _make_sc_agg(d):
    @functools.partial(
        pl.kernel,
        out_type=jax.ShapeDtypeStruct((NCORE, NP, d), jnp.float32),
        mesh=_sc_mesh(),
        scratch_types=[
            pltpu.VMEM((HL, CH), jnp.int32),
            pltpu.VMEM((HL, CH), jnp.int32),
            pltpu.VMEM((2, CH, d), jnp.float32),
            pltpu.VMEM_SHARED((NP, d), jnp.float32),
            pltpu.SemaphoreType.DMA,
            pltpu.SemaphoreType.DMA,
        ],
    )
    def agg(gs_hbm, src_hbm, dst_hbm, zr_hbm, out_hbm, src_v, dst_v, rows_v,
            acc_sh, gsem, ssem):
        cid = lax.axis_index("c")
        sid = lax.axis_index("s")
        wid = cid * NSUB + sid
        pltpu.sync_copy(zr_hbm, acc_sh.at[pl.ds(sid * RPS, RPS)])
        plsc.subcore_barrier()

        def fire_gather(j, k):
            pltpu.async_copy(gs_hbm.at[src_v.at[j]], rows_v.at[k], gsem)

        def drain_gather():
            pltpu.make_async_copy(gs_hbm.at[pl.ds(0, CH)], rows_v.at[0],
                                  gsem).wait()

        def scatter(j, k):
            pltpu.async_copy(rows_v.at[k], acc_sh.at[dst_v.at[j]],
                             ssem, add=True).wait()

        nst = jnp.where(cid == 0, NST0, NST1)

        def stage_body(st, carry):
            pltpu.sync_copy(src_hbm.at[wid, pl.ds(st * HL, HL)], src_v)
            pltpu.sync_copy(dst_hbm.at[wid, pl.ds(st * HL, HL)], dst_v)
            fire_gather(0, 0)

            def body(i, c2):
                jb = i * 2
                drain_gather()
                fire_gather(jb + 1, 1)
                scatter(jb, 0)
                drain_gather()
                @pl.when(i < HL // 2 - 1)
                def _():
                    fire_gather(jb + 2, 0)
                scatter(jb + 1, 1)
                return c2

            lax.fori_loop(0, HL // 2, body, 0)
            return carry

        lax.fori_loop(0, nst, stage_body, 0)

        plsc.subcore_barrier()
        pltpu.sync_copy(acc_sh.at[pl.ds(sid * RPS, RPS)],
                        out_hbm.at[cid, pl.ds(sid * RPS, RPS)])

    return agg


_sc_agg_d = _make_sc_agg(D)
_sc_agg_c = _make_sc_agg(DC)


def _tc0_body(degp_ref, x_ref, w_ref, dinv_ref, gs_ref):
    deg = degp_ref[0, :, 0:1] + degp_ref[1, :, 0:1]
    rid = lax.broadcasted_iota(jnp.int32, (NP, 1), 0)
    deg = jnp.where(rid < N, deg + 1.0, deg)
    dinv = jnp.where(deg > 0, lax.rsqrt(deg), 0.0)
    dinv_ref[...] = dinv
    h = jnp.dot(x_ref[...], w_ref[...], preferred_element_type=jnp.float32)
    gs_ref[...] = jnp.zeros((NP, D), jnp.float32)
    gs_ref[0:N, :] = h * dinv[0:N]


def _tc0(degp, x, w1):
    return pl.pallas_call(
        _tc0_body,
        out_shape=(jax.ShapeDtypeStruct((NP, 1), jnp.float32),
                   jax.ShapeDtypeStruct((NP, D), jnp.float32)),
    )(degp, x, w1)


def _make_tc_mid(dout, relu):
    def body(aggp_ref, gs_ref, dinv_ref, b_ref, w_ref, out_ref):
        dinv = dinv_ref[...]
        t = aggp_ref[0] + aggp_ref[1] + gs_ref[...]
        h = t * dinv + b_ref[...]
        if relu:
            h = jnp.maximum(h, 0.0)
        g = jnp.dot(h, w_ref[...], preferred_element_type=jnp.float32)
        rid = lax.broadcasted_iota(jnp.int32, (NP, 1), 0)
        out_ref[...] = jnp.where(rid < N, g * dinv, 0.0)

    def call(aggp, gs, dinv, b, w):
        return pl.pallas_call(
            body,
            out_shape=jax.ShapeDtypeStruct((NP, dout), jnp.float32),
        )(aggp, gs, dinv, b, w)

    return call


_tc_mid = _make_tc_mid(D, relu=False)
_tc4 = _make_tc_mid(DC, relu=True)


def _tc5_body(aggp_ref, gs_ref, dinv_ref, b_ref, out_ref):
    t = aggp_ref[0] + aggp_ref[1] + gs_ref[...]
    h = t * dinv_ref[...] + b_ref[...]
    v = h[0:N, 0:NC_REAL]
    m = jnp.max(v, axis=0, keepdims=True)
    e = jnp.exp(v - m)
    out_ref[...] = e / jnp.sum(e, axis=0, keepdims=True)


def _tc5(aggp, gs, dinv, b):
    return pl.pallas_call(
        _tc5_body,
        out_shape=jax.ShapeDtypeStruct((N, NC_REAL), jnp.float32),
    )(aggp, gs, dinv, b)


@jax.jit
def kernel(x, edge_index, W1, b1, Wi, bi, W2, b2):
    src = edge_index[0].astype(jnp.int32)
    dst = edge_index[1].astype(jnp.int32)
    pad = jnp.full((EPAD - E,), PN, jnp.int32)

    def layout(idx):
        idx = jnp.concatenate([idx, pad])
        c0 = idx[:E0].reshape(NSUB, CCH0, CH)
        c1 = idx[E0:].reshape(NSUB, CCH1, CH)
        c1 = jnp.pad(c1, ((0, 0), (0, CCHM - CCH1), (0, 0)),
                     constant_values=PN)
        return jnp.concatenate([c0, c1], axis=0)

    src3 = layout(src)
    dst3 = layout(dst)

    zr_d = jnp.zeros((RPS, D), jnp.float32)
    zr_c = jnp.zeros((RPS, DC), jnp.float32)
    ones_m = jnp.ones((NP, D), jnp.float32)

    w2p = jnp.zeros((D, DC), jnp.float32).at[:, :NC_REAL].set(W2)
    b1r = jnp.reshape(b1, (1, D))
    bir = jnp.reshape(bi, (1, D))
    b2p = jnp.zeros((1, DC), jnp.float32).at[0, :NC_REAL].set(b2)

    degp = _sc_agg_d(ones_m, src3, dst3, zr_d)
    dinv, gs = _tc0(degp, x, W1)

    aggp = _sc_agg_d(gs, src3, dst3, zr_d)
    gs = _tc_mid(aggp, gs, dinv, b1r, Wi)
    for _ in range(2):
        aggp = _sc_agg_d(gs, src3, dst3, zr_d)
        gs = _tc_mid(aggp, gs, dinv, bir, Wi)
    aggp = _sc_agg_d(gs, src3, dst3, zr_d)
    gs = _tc4(aggp, gs, dinv, bir, w2p)

    aggp = _sc_agg_c(gs, src3, dst3, zr_c)
    return _tc5(aggp, gs, dinv, b2p)


# speedup vs baseline: 7.3839x; 1.1588x over previous
import functools

import jax
import jax.numpy as jnp
from jax import lax
from jax.experimental import pallas as pl
from jax.experimental.pallas import tpu as pltpu
from jax.experimental.pallas import tpu_sc as plsc

N = 10000
E = 320000
D = 128
DC = 128
NC_REAL = 40
NP = 10240
PN = N
NCORE = 2
NSUB = 16
NW = NCORE * NSUB
CH = 128
HL = 32
NST0 = 4
NST1 = 1
CCH0 = NST0 * HL
CCH1 = NST1 * HL
CCHM = max(CCH0, CCH1)
EPAD = NSUB * (CCH0 + CCH1) * CH
E0 = NSUB * CCH0 * CH
RPS = NP // NSUB


ZB = 40


def _sc_mesh():
    return plsc.VectorSubcoreMesh(core_axis_name="c", subcore_axis_name="s")


def _zero_acc(zb_v, acc_sh, sid, d, sem):
    def zrow(r, c):
        for k in range(d // 16):
            zb_v[r, pl.ds(k * 16, 16)] = jnp.zeros((16,), jnp.float32)
        return c

    lax.fori_loop(0, ZB, zrow, 0)
    descs = [pltpu.async_copy(zb_v, acc_sh.at[pl.ds(sid * RPS + t * ZB, ZB)],
                              sem) for t in range(RPS // ZB)]
    for de in descs:
        de.wait()


def _make_sc_agg(d):
    @functools.partial(
        pl.kernel,
        out_type=jax.ShapeDtypeStruct((NCORE, NP, d), jnp.float32),
        mesh=_sc_mesh(),
        scratch_types=[
            pltpu.VMEM((HL, CH), jnp.int32),
            pltpu.VMEM((HL, CH), jnp.int32),
            pltpu.VMEM((2, CH, d), jnp.float32),
            pltpu.VMEM((ZB, d), jnp.float32),
            pltpu.VMEM_SHARED((NP, d), jnp.float32),
            pltpu.SemaphoreType.DMA,
            pltpu.SemaphoreType.DMA,
        ],
    )
    def agg(gs_hbm, src_hbm, dst_hbm, out_hbm, src_v, dst_v, rows_v,
            zb_v, acc_sh, gsem, ssem):
        cid = lax.axis_index("c")
        sid = lax.axis_index("s")
        wid = cid * NSUB + sid
        _zero_acc(zb_v, acc_sh, sid, d, ssem)
        plsc.subcore_barrier()

        def fire_gather(j, k):
            pltpu.async_copy(gs_hbm.at[src_v.at[j]], rows_v.at[k], gsem)

        def drain_gather():
            pltpu.make_async_copy(gs_hbm.at[pl.ds(0, CH)], rows_v.at[0],
                                  gsem).wait()

        def scatter(j, k):
            pltpu.async_copy(rows_v.at[k], acc_sh.at[dst_v.at[j]],
                             ssem, add=True).wait()

        nst = jnp.where(cid == 0, NST0, NST1)

        def stage_body(st, carry):
            pltpu.sync_copy(src_hbm.at[wid, pl.ds(st * HL, HL)], src_v)
            pltpu.sync_copy(dst_hbm.at[wid, pl.ds(st * HL, HL)], dst_v)
            fire_gather(0, 0)

            def body(i, c2):
                jb = i * 2
                drain_gather()
                fire_gather(jb + 1, 1)
                scatter(jb, 0)
                drain_gather()
                @pl.when(i < HL // 2 - 1)
                def _():
                    fire_gather(jb + 2, 0)
                scatter(jb + 1, 1)
                return c2

            lax.fori_loop(0, HL // 2, body, 0)
            return carry

        lax.fori_loop(0, nst, stage_body, 0)

        plsc.subcore_barrier()
        pltpu.sync_copy(acc_sh.at[pl.ds(sid * RPS, RPS)],
                        out_hbm.at[cid, pl.ds(sid * RPS, RPS)])

    return agg


_sc_agg_d = _make_sc_agg(D)
_sc_agg_c = _make_sc_agg(DC)


@functools.partial(
    pl.kernel,
    out_type=jax.ShapeDtypeStruct((NCORE, NP, D), jnp.float32),
    mesh=_sc_mesh(),
    scratch_types=[
        pltpu.VMEM((HL, CH), jnp.int32),
        pltpu.VMEM((CH, D), jnp.float32),
        pltpu.VMEM((ZB, D), jnp.float32),
        pltpu.VMEM_SHARED((NP, D), jnp.float32),
        pltpu.SemaphoreType.DMA,
    ],
)
def _sc_deg(dst_hbm, ones_hbm, out_hbm, dst_v, ones_v, zb_v, acc_sh, ssem):
    cid = lax.axis_index("c")
    sid = lax.axis_index("s")
    wid = cid * NSUB + sid
    _zero_acc(zb_v, acc_sh, sid, D, ssem)
    pltpu.sync_copy(ones_hbm, ones_v)
    plsc.subcore_barrier()

    nst = jnp.where(cid == 0, NST0, NST1)

    def stage_body(st, carry):
        pltpu.sync_copy(dst_hbm.at[wid, pl.ds(st * HL, HL)], dst_v)

        def body(i, c2):
            jb = i * 2
            da = pltpu.async_copy(ones_v, acc_sh.at[dst_v.at[jb]], ssem,
                                  add=True)
            db = pltpu.async_copy(ones_v, acc_sh.at[dst_v.at[jb + 1]], ssem,
                                  add=True)
            da.wait()
            db.wait()
            return c2

        lax.fori_loop(0, HL // 2, body, 0)
        return carry

    lax.fori_loop(0, nst, stage_body, 0)
    plsc.subcore_barrier()
    pltpu.sync_copy(acc_sh.at[pl.ds(sid * RPS, RPS)],
                    out_hbm.at[cid, pl.ds(sid * RPS, RPS)])


def _tc0_body(degp_ref, x_ref, w_ref, dinv_ref, gs_ref):
    deg = degp_ref[0, :, 0:1] + degp_ref[1, :, 0:1]
    rid = lax.broadcasted_iota(jnp.int32, (NP, 1), 0)
    deg = jnp.where(rid < N, deg + 1.0, deg)
    dinv = jnp.where(deg > 0, lax.rsqrt(deg), 0.0)
    dinv_ref[...] = dinv
    h = jnp.dot(x_ref[...], w_ref[...], preferred_element_type=jnp.float32)
    gs_ref[...] = jnp.zeros((NP, D), jnp.float32)
    gs_ref[0:N, :] = h * dinv[0:N]


def _tc0(degp, x, w1):
    return pl.pallas_call(
        _tc0_body,
        out_shape=(jax.ShapeDtypeStruct((NP, 1), jnp.float32),
                   jax.ShapeDtypeStruct((NP, D), jnp.float32)),
    )(degp, x, w1)


def _make_tc_mid(dout, relu):
    def body(aggp_ref, gs_ref, dinv_ref, b_ref, w_ref, out_ref):
        dinv = dinv_ref[...]
        t = aggp_ref[0] + aggp_ref[1] + gs_ref[...]
        h = t * dinv + b_ref[...]
        if relu:
            h = jnp.maximum(h, 0.0)
        g = jnp.dot(h, w_ref[...], preferred_element_type=jnp.float32)
        rid = lax.broadcasted_iota(jnp.int32, (NP, 1), 0)
        out_ref[...] = jnp.where(rid < N, g * dinv, 0.0)

    def call(aggp, gs, dinv, b, w):
        return pl.pallas_call(
            body,
            out_shape=jax.ShapeDtypeStruct((NP, dout), jnp.float32),
        )(aggp, gs, dinv, b, w)

    return call


_tc_mid = _make_tc_mid(D, relu=False)
_tc4 = _make_tc_mid(DC, relu=True)


def _tc5_body(aggp_ref, gs_ref, dinv_ref, b_ref, out_ref):
    t = aggp_ref[0] + aggp_ref[1] + gs_ref[...]
    h = t * dinv_ref[...] + b_ref[...]
    v = h[0:N, 0:NC_REAL]
    m = jnp.max(v, axis=0, keepdims=True)
    e = jnp.exp(v - m)
    out_ref[...] = e / jnp.sum(e, axis=0, keepdims=True)


def _tc5(aggp, gs, dinv, b):
    return pl.pallas_call(
        _tc5_body,
        out_shape=jax.ShapeDtypeStruct((N, NC_REAL), jnp.float32),
    )(aggp, gs, dinv, b)


@jax.jit
def kernel(x, edge_index, W1, b1, Wi, bi, W2, b2):
    src = edge_index[0].astype(jnp.int32)
    dst = edge_index[1].astype(jnp.int32)
    pad = jnp.full((EPAD - E,), PN, jnp.int32)

    def layout(idx):
        idx = jnp.concatenate([idx, pad])
        c0 = idx[:E0].reshape(NSUB, CCH0, CH)
        c1 = idx[E0:].reshape(NSUB, CCH1, CH)
        c1 = jnp.pad(c1, ((0, 0), (0, CCHM - CCH1), (0, 0)),
                     constant_values=PN)
        return jnp.concatenate([c0, c1], axis=0)

    src3 = layout(src)
    dst3 = layout(dst)

    ones_m = jnp.ones((CH, D), jnp.float32)

    w2p = jnp.zeros((D, DC), jnp.float32).at[:, :NC_REAL].set(W2)
    b1r = jnp.reshape(b1, (1, D))
    bir = jnp.reshape(bi, (1, D))
    b2p = jnp.zeros((1, DC), jnp.float32).at[0, :NC_REAL].set(b2)

    degp = _sc_deg(dst3, ones_m)
    dinv, gs = _tc0(degp, x, W1)

    aggp = _sc_agg_d(gs, src3, dst3)
    gs = _tc_mid(aggp, gs, dinv, b1r, Wi)
    for _ in range(2):
        aggp = _sc_agg_d(gs, src3, dst3)
        gs = _tc_mid(aggp, gs, dinv, bir, Wi)
    aggp = _sc_agg_d(gs, src3, dst3)
    gs = _tc4(aggp, gs, dinv, bir, w2p)

    aggp = _sc_agg_c(gs, src3, dst3)
    return _tc5(aggp, gs, dinv, b2p)


# speedup vs baseline: 7.4214x; 1.0051x over previous
import functools

import jax
import jax.numpy as jnp
from jax import lax
from jax.experimental import pallas as pl
from jax.experimental.pallas import tpu as pltpu
from jax.experimental.pallas import tpu_sc as plsc

N = 10000
E = 320000
D = 128
DC = 128
NC_REAL = 40
NP = 10240
PN = N
NCORE = 2
NSUB = 16
NW = NCORE * NSUB
CH = 128
HL = 32
NST = 5
PC0 = 28
PC1 = 4
CCHM = NST * HL
EPAD = NSUB * NST * (PC0 + PC1) * CH
E0 = NSUB * NST * PC0 * CH
RPS = NP // NSUB


ZB = 40


def _sc_mesh():
    return plsc.VectorSubcoreMesh(core_axis_name="c", subcore_axis_name="s")


def _zero_acc(zb_v, acc_sh, sid, d, sem):
    def zrow(r, c):
        for k in range(d // 16):
            zb_v[r, pl.ds(k * 16, 16)] = jnp.zeros((16,), jnp.float32)
        return c

    lax.fori_loop(0, ZB, zrow, 0)
    descs = [pltpu.async_copy(zb_v, acc_sh.at[pl.ds(sid * RPS + t * ZB, ZB)],
                              sem) for t in range(RPS // ZB)]
    for de in descs:
        de.wait()


def _make_sc_agg(d):
    @functools.partial(
        pl.kernel,
        out_type=jax.ShapeDtypeStruct((NCORE, NP, d), jnp.float32),
        mesh=_sc_mesh(),
        scratch_types=[
            pltpu.VMEM((HL, CH), jnp.int32),
            pltpu.VMEM((HL, CH), jnp.int32),
            pltpu.VMEM((2, CH, d), jnp.float32),
            pltpu.VMEM((ZB, d), jnp.float32),
            pltpu.VMEM_SHARED((NP, d), jnp.float32),
            pltpu.SemaphoreType.DMA,
            pltpu.SemaphoreType.DMA,
        ],
    )
    def agg(gs_hbm, src_hbm, dst_hbm, out_hbm, src_v, dst_v, rows_v,
            zb_v, acc_sh, gsem, ssem):
        cid = lax.axis_index("c")
        sid = lax.axis_index("s")
        wid = cid * NSUB + sid
        _zero_acc(zb_v, acc_sh, sid, d, ssem)
        plsc.subcore_barrier()

        def fire_gather(j, k):
            pltpu.async_copy(gs_hbm.at[src_v.at[j]], rows_v.at[k], gsem)

        def drain_gather():
            pltpu.make_async_copy(gs_hbm.at[pl.ds(0, CH)], rows_v.at[0],
                                  gsem).wait()

        def scatter(j, k):
            pltpu.async_copy(rows_v.at[k], acc_sh.at[dst_v.at[j]],
                             ssem, add=True).wait()

        def chunk_loop(trip):
            fire_gather(0, 0)

            def body(i, c2):
                jb = i * 2
                drain_gather()
                fire_gather(jb + 1, 1)
                scatter(jb, 0)
                drain_gather()
                @pl.when(i < trip - 1)
                def _():
                    fire_gather(jb + 2, 0)
                scatter(jb + 1, 1)
                return c2

            lax.fori_loop(0, trip, body, 0)

        def stage_body(st, carry):
            pltpu.sync_copy(src_hbm.at[wid, pl.ds(st * HL, HL)], src_v)
            pltpu.sync_copy(dst_hbm.at[wid, pl.ds(st * HL, HL)], dst_v)

            @pl.when(cid == 0)
            def _():
                chunk_loop(PC0 // 2)

            @pl.when(cid == 1)
            def _():
                chunk_loop(PC1 // 2)

            return carry

        lax.fori_loop(0, NST, stage_body, 0)

        plsc.subcore_barrier()
        pltpu.sync_copy(acc_sh.at[pl.ds(sid * RPS, RPS)],
                        out_hbm.at[cid, pl.ds(sid * RPS, RPS)])

    return agg


_sc_agg_d = _make_sc_agg(D)
_sc_agg_c = _make_sc_agg(DC)


@functools.partial(
    pl.kernel,
    out_type=jax.ShapeDtypeStruct((NCORE, NP, D), jnp.float32),
    mesh=_sc_mesh(),
    scratch_types=[
        pltpu.VMEM((HL, CH), jnp.int32),
        pltpu.VMEM((CH, D), jnp.float32),
        pltpu.VMEM((ZB, D), jnp.float32),
        pltpu.VMEM_SHARED((NP, D), jnp.float32),
        pltpu.SemaphoreType.DMA,
    ],
)
def _sc_deg(dst_hbm, ones_hbm, out_hbm, dst_v, ones_v, zb_v, acc_sh, ssem):
    cid = lax.axis_index("c")
    sid = lax.axis_index("s")
    wid = cid * NSUB + sid
    _zero_acc(zb_v, acc_sh, sid, D, ssem)
    pltpu.sync_copy(ones_hbm, ones_v)
    plsc.subcore_barrier()

    trip = jnp.where(cid == 0, PC0 // 2, PC1 // 2)

    def stage_body(st, carry):
        pltpu.sync_copy(dst_hbm.at[wid, pl.ds(st * HL, HL)], dst_v)

        def body(i, c2):
            jb = i * 2
            da = pltpu.async_copy(ones_v, acc_sh.at[dst_v.at[jb]], ssem,
                                  add=True)
            db = pltpu.async_copy(ones_v, acc_sh.at[dst_v.at[jb + 1]], ssem,
                                  add=True)
            da.wait()
            db.wait()
            return c2

        lax.fori_loop(0, trip, body, 0)
        return carry

    lax.fori_loop(0, NST, stage_body, 0)
    plsc.subcore_barrier()
    pltpu.sync_copy(acc_sh.at[pl.ds(sid * RPS, RPS)],
                    out_hbm.at[cid, pl.ds(sid * RPS, RPS)])


def _tc0_body(degp_ref, x_ref, w_ref, dinv_ref, gs_ref):
    deg = degp_ref[0, :, 0:1] + degp_ref[1, :, 0:1]
    rid = lax.broadcasted_iota(jnp.int32, (NP, 1), 0)
    deg = jnp.where(rid < N, deg + 1.0, deg)
    dinv = jnp.where(deg > 0, lax.rsqrt(deg), 0.0)
    dinv_ref[...] = dinv
    h = jnp.dot(x_ref[...], w_ref[...], preferred_element_type=jnp.float32)
    gs_ref[...] = jnp.zeros((NP, D), jnp.float32)
    gs_ref[0:N, :] = h * dinv[0:N]


def _tc0(degp, x, w1):
    return pl.pallas_call(
        _tc0_body,
        out_shape=(jax.ShapeDtypeStruct((NP, 1), jnp.float32),
                   jax.ShapeDtypeStruct((NP, D), jnp.float32)),
    )(degp, x, w1)


def _make_tc_mid(dout, relu):
    def body(aggp_ref, gs_ref, dinv_ref, b_ref, w_ref, out_ref):
        dinv = dinv_ref[...]
        t = aggp_ref[0] + aggp_ref[1] + gs_ref[...]
        h = t * dinv + b_ref[...]
        if relu:
            h = jnp.maximum(h, 0.0)
        g = jnp.dot(h, w_ref[...], preferred_element_type=jnp.float32)
        rid = lax.broadcasted_iota(jnp.int32, (NP, 1), 0)
        out_ref[...] = jnp.where(rid < N, g * dinv, 0.0)

    def call(aggp, gs, dinv, b, w):
        return pl.pallas_call(
            body,
            out_shape=jax.ShapeDtypeStruct((NP, dout), jnp.float32),
        )(aggp, gs, dinv, b, w)

    return call


_tc_mid = _make_tc_mid(D, relu=False)
_tc4 = _make_tc_mid(DC, relu=True)


def _tc5_body(aggp_ref, gs_ref, dinv_ref, b_ref, out_ref):
    t = aggp_ref[0] + aggp_ref[1] + gs_ref[...]
    h = t * dinv_ref[...] + b_ref[...]
    v = h[0:N, 0:NC_REAL]
    m = jnp.max(v, axis=0, keepdims=True)
    e = jnp.exp(v - m)
    out_ref[...] = e / jnp.sum(e, axis=0, keepdims=True)


def _tc5(aggp, gs, dinv, b):
    return pl.pallas_call(
        _tc5_body,
        out_shape=jax.ShapeDtypeStruct((N, NC_REAL), jnp.float32),
    )(aggp, gs, dinv, b)


@jax.jit
def kernel(x, edge_index, W1, b1, Wi, bi, W2, b2):
    src = edge_index[0].astype(jnp.int32)
    dst = edge_index[1].astype(jnp.int32)
    pad = jnp.full((EPAD - E,), PN, jnp.int32)

    def layout(idx):
        idx = jnp.concatenate([idx, pad])
        c0 = idx[:E0].reshape(NSUB, NST, PC0, CH)
        c0 = jnp.pad(c0, ((0, 0), (0, 0), (0, HL - PC0), (0, 0)),
                     constant_values=PN).reshape(NSUB, CCHM, CH)
        c1 = idx[E0:].reshape(NSUB, NST, PC1, CH)
        c1 = jnp.pad(c1, ((0, 0), (0, 0), (0, HL - PC1), (0, 0)),
                     constant_values=PN).reshape(NSUB, CCHM, CH)
        return jnp.concatenate([c0, c1], axis=0)

    src3 = layout(src)
    dst3 = layout(dst)

    ones_m = jnp.ones((CH, D), jnp.float32)

    w2p = jnp.zeros((D, DC), jnp.float32).at[:, :NC_REAL].set(W2)
    b1r = jnp.reshape(b1, (1, D))
    bir = jnp.reshape(bi, (1, D))
    b2p = jnp.zeros((1, DC), jnp.float32).at[0, :NC_REAL].set(b2)

    degp = _sc_deg(dst3, ones_m)
    dinv, gs = _tc0(degp, x, W1)

    aggp = _sc_agg_d(gs, src3, dst3)
    gs = _tc_mid(aggp, gs, dinv, b1r, Wi)
    for _ in range(2):
        aggp = _sc_agg_d(gs, src3, dst3)
        gs = _tc_mid(aggp, gs, dinv, bir, Wi)
    aggp = _sc_agg_d(gs, src3, dst3)
    gs = _tc4(aggp, gs, dinv, bir, w2p)

    aggp = _sc_agg_c(gs, src3, dst3)
    return _tc5(aggp, gs, dinv, b2p)
